# baseline (speedup 1.0000x reference)
import functools

import jax
import jax.numpy as jnp
from jax import lax
from jax.experimental import pallas as pl
from jax.experimental.pallas import tpu as pltpu
from jax.experimental.pallas import tpu_sc as plsc

C = 150
HW = 224 * 224
N = 4
NPIX = N * HW

NC = 2
NS = 16
NW = NC * NS
B_PER_W = NPIX // NW
LANES = 16
N_VEC = B_PER_W // LANES
G = 128
NG = B_PER_W // G
ALPHA_PAD = 160


@functools.partial(
    pl.kernel,
    mesh=plsc.VectorSubcoreMesh(core_axis_name="c", subcore_axis_name="s"),
    out_type=[
        jax.ShapeDtypeStruct((NPIX,), jnp.float32),
        jax.ShapeDtypeStruct((NPIX,), jnp.float32),
    ],
    scratch_types=[
        pltpu.VMEM((B_PER_W,), jnp.int32),
        pltpu.VMEM((B_PER_W,), jnp.int32),
        pltpu.VMEM((B_PER_W,), jnp.float32),
        pltpu.VMEM((B_PER_W,), jnp.float32),
        pltpu.VMEM((ALPHA_PAD,), jnp.float32),
        pltpu.SemaphoreType.DMA,
    ],
    compiler_params=pltpu.CompilerParams(needs_layout_passes=False),
)
def _sc_gather(preds_hbm, t_hbm, alpha_hbm, xt_hbm, a_hbm,
               t_v, idx_v, xt_v, a_v, alpha_v, sem):
    wid = lax.axis_index("s") * NC + lax.axis_index("c")
    base = wid * B_PER_W
    n = base // HW
    off0 = n * C * HW + (base - n * HW)
    pltpu.sync_copy(t_hbm.at[pl.ds(base, B_PER_W)], t_v)
    pltpu.sync_copy(alpha_hbm, alpha_v)

    def body(i, carry):
        t16 = t_v[pl.ds(i * LANES, LANES)]
        off = off0 + i * LANES
        idx_v[pl.ds(i * LANES, LANES)] = (
            t16 * HW + off + lax.iota(jnp.int32, LANES))
        a_v[pl.ds(i * LANES, LANES)] = plsc.load_gather(alpha_v, [t16])
        return carry

    lax.fori_loop(0, N_VEC, body, 0)

    copies = [
        pltpu.make_async_copy(
            preds_hbm.at[idx_v.at[pl.ds(j * G, G)]],
            xt_v.at[pl.ds(j * G, G)],
            sem,
        )
        for j in range(NG)
    ]
    for cp in copies:
        cp.start()
    for cp in copies:
        cp.wait()

    pltpu.sync_copy(xt_v, xt_hbm.at[pl.ds(base, B_PER_W)])
    pltpu.sync_copy(a_v, a_hbm.at[pl.ds(base, B_PER_W)])


PIX_BLOCK = 6272
N_PIX_BLOCKS = HW // PIX_BLOCK


def _lse_kernel(x_ref, lse_ref):
    x = x_ref[0]
    m = jnp.max(x, axis=0)
    s = jnp.sum(jnp.exp(x - m), axis=0)
    lse_ref[...] = (m + jnp.log(s)).reshape(1, 1, PIX_BLOCK)


def _lse(x):
    return pl.pallas_call(
        _lse_kernel,
        grid=(N, N_PIX_BLOCKS),
        in_specs=[pl.BlockSpec((1, C, PIX_BLOCK), lambda n, b: (n, 0, b))],
        out_specs=pl.BlockSpec((1, 1, PIX_BLOCK), lambda n, b: (n, 0, b)),
        out_shape=jax.ShapeDtypeStruct((N, 1, HW), jnp.float32),
    )(x)


ROWS = NPIX // 128


def _combine_kernel(xt_ref, a_ref, lse_ref, acc_ref):
    log_pt = xt_ref[...] - lse_ref[...]
    pt = jnp.exp(log_pt)
    q = 1.0 - pt
    loss = -a_ref[...] * q * q * log_pt
    acc_ref[...] = jnp.sum(loss).reshape(1, 1)


def _combine(xt, a, lse):
    return pl.pallas_call(
        _combine_kernel,
        out_shape=jax.ShapeDtypeStruct((1, 1), jnp.float32),
    )(xt.reshape(ROWS, 128), a.reshape(ROWS, 128), lse.reshape(ROWS, 128))


def kernel(preds, targets, alpha):
    x = preds.reshape(N, C, HW)
    t_flat = targets.reshape(NPIX).astype(jnp.int32)
    alpha_pad = jnp.concatenate(
        [alpha.reshape(C), jnp.zeros((ALPHA_PAD - C,), jnp.float32)])

    xt, a = _sc_gather(preds.reshape(N * C * HW), t_flat, alpha_pad)
    lse = _lse(x)
    acc = _combine(xt, a, lse)
    return acc[0, 0] / NPIX


# speedup vs baseline: 5.3543x; 5.3543x over previous
import jax
import jax.numpy as jnp
from jax.experimental import pallas as pl

C = 150
H = 224
W = 224
N = 4
HBLK = 56
N_HBLK = H // HBLK


def _focal_kernel(x_ref, t_ref, alpha_ref, acc_ref):
    n = pl.program_id(0)
    b = pl.program_id(1)

    @pl.when((n == 0) & (b == 0))
    def _():
        acc_ref[...] = jnp.zeros_like(acc_ref)

    x = x_ref[0]
    t = t_ref[...]
    alpha = alpha_ref[...].reshape(C, 1, 1)

    cls = jax.lax.broadcasted_iota(jnp.int32, x.shape, 0)
    mask = (cls == t).astype(jnp.float32)
    xt = jnp.sum(mask * x, axis=0)
    a = jnp.sum(mask * alpha, axis=0)

    m = jnp.max(x, axis=0)
    s = jnp.sum(jnp.exp(x - m), axis=0)
    log_pt = xt - m - jnp.log(s)
    pt = jnp.exp(log_pt)
    q = 1.0 - pt
    loss = -a * q * q * log_pt
    acc_ref[...] += jnp.sum(loss).reshape(1, 1)


def kernel(preds, targets, alpha):
    acc = pl.pallas_call(
        _focal_kernel,
        grid=(N, N_HBLK),
        in_specs=[
            pl.BlockSpec((1, C, HBLK, W), lambda n, b: (n, 0, b, 0)),
            pl.BlockSpec((1, HBLK, W), lambda n, b: (n, b, 0)),
            pl.BlockSpec((C, 1), lambda n, b: (0, 0)),
        ],
        out_specs=pl.BlockSpec((1, 1), lambda n, b: (0, 0)),
        out_shape=jax.ShapeDtypeStruct((1, 1), jnp.float32),
    )(preds, targets.astype(jnp.int32), alpha)

    return acc[0, 0] / (N * H * W)
